# baseline (speedup 1.0000x reference)
import jax
import jax.numpy as jnp
from jax.experimental import pallas as pl
from jax.experimental.pallas import tpu as pltpu

EPS = 1e-8

DB = 65536
QB = 128
CHUNK = 8192
NCHUNK = DB // CHUNK


def _lookup_kernel(q_ref, t_ref, x_ref, s_ref, o_ref, e_ref, xn_ref, mc_ref):
    @pl.when(pl.program_id(0) == 0)
    def _prologue():
        def norms_minmax(c, carry):
            mn, mx = carry
            xc = x_ref[:, pl.ds(c * CHUNK, CHUNK)]
            xn_ref[:, pl.ds(c * CHUNK, CHUNK)] = jnp.sum(
                xc * xc, axis=0, keepdims=True)
            sc = s_ref[:, pl.ds(c * CHUNK, CHUNK)]
            return jnp.minimum(mn, jnp.min(sc)), jnp.maximum(mx, jnp.max(sc))

        mn, mx = jax.lax.fori_loop(
            0, NCHUNK, norms_minmax, (jnp.inf, -jnp.inf))
        rng = mx - mn

        def norm_metric(c, _):
            sc = s_ref[:, pl.ds(c * CHUNK, CHUNK)]
            mc_ref[:, pl.ds(c * CHUNK, CHUNK)] = (sc - mn) / rng
            return 0

        jax.lax.fori_loop(0, NCHUNK, norm_metric, 0)

    temp = t_ref[0, 0] + EPS
    q = q_ref[...]
    q2 = q + q
    qn = jnp.sum(q * q, axis=1, keepdims=True)

    z = jnp.zeros((QB, 1), jnp.float32)
    for c in range(NCHUNK):
        xc = x_ref[:, pl.ds(c * CHUNK, CHUNK)]
        dot2 = jax.lax.dot_general(
            q2, xc, (((1,), (0,)), ((), ())),
            preferred_element_type=jnp.float32)
        neg_d = dot2 - (qn + xn_ref[:, pl.ds(c * CHUNK, CHUNK)])
        e = jnp.exp(neg_d / temp)
        e_ref[:, pl.ds(c * CHUNK, CHUNK)] = e
        z = z + jnp.sum(e, axis=1, keepdims=True)

    th = 0.001 * z

    acc = jnp.zeros((QB, 1), jnp.float32)
    for c in range(NCHUNK):
        e = e_ref[:, pl.ds(c * CHUNK, CHUNK)]
        we = jnp.where(e >= th, e, 0.0)
        mc = mc_ref[:, pl.ds(c * CHUNK, CHUNK)]
        acc = acc + jnp.sum(we * mc, axis=1, keepdims=True)

    o_ref[...] = acc / z


@jax.jit
def kernel(query_vectors, temperature, indices, s1_metric):
    orig_dtype = query_vectors.dtype
    batch = query_vectors.shape[0]
    q = query_vectors.astype(jnp.float32)
    x_t = indices.astype(jnp.float32).T
    s = s1_metric.astype(jnp.float32).reshape(1, DB)
    t = temperature.astype(jnp.float32).reshape(1, 1)

    grid = (batch // QB,)
    out = pl.pallas_call(
        _lookup_kernel,
        grid=grid,
        in_specs=[
            pl.BlockSpec((QB, 3), lambda i: (i, 0)),
            pl.BlockSpec((1, 1), lambda i: (0, 0)),
            pl.BlockSpec((3, DB), lambda i: (0, 0)),
            pl.BlockSpec((1, DB), lambda i: (0, 0)),
        ],
        out_specs=pl.BlockSpec((QB, 1), lambda i: (i, 0)),
        out_shape=jax.ShapeDtypeStruct((batch, 1), jnp.float32),
        scratch_shapes=[
            pltpu.VMEM((QB, DB), jnp.float32),
            pltpu.VMEM((1, DB), jnp.float32),
            pltpu.VMEM((1, DB), jnp.float32),
        ],
    )(q, t, x_t, s)
    return out.reshape(batch).astype(orig_dtype)


# speedup vs baseline: 1.1627x; 1.1627x over previous
import jax
import jax.numpy as jnp
from jax.experimental import pallas as pl
from jax.experimental.pallas import tpu as pltpu

EPS = 1e-8

DB = 65536
QB = 128
CHUNK = 16384
NCHUNK = DB // CHUNK


def _lookup_kernel(q_ref, t_ref, x_ref, s_ref, o_ref, e_ref, xn_ref, mc_ref):
    @pl.when(pl.program_id(0) == 0)
    def _prologue():
        def norms_minmax(c, carry):
            mn, mx = carry
            xc = x_ref[:, pl.ds(c * CHUNK, CHUNK)]
            xn_ref[:, pl.ds(c * CHUNK, CHUNK)] = jnp.sum(
                xc * xc, axis=0, keepdims=True)
            sc = s_ref[:, pl.ds(c * CHUNK, CHUNK)]
            return jnp.minimum(mn, jnp.min(sc)), jnp.maximum(mx, jnp.max(sc))

        mn, mx = jax.lax.fori_loop(
            0, NCHUNK, norms_minmax, (jnp.inf, -jnp.inf))
        rng = mx - mn

        def norm_metric(c, _):
            sc = s_ref[:, pl.ds(c * CHUNK, CHUNK)]
            mc_ref[:, pl.ds(c * CHUNK, CHUNK)] = (sc - mn) / rng
            return 0

        jax.lax.fori_loop(0, NCHUNK, norm_metric, 0)

    temp = t_ref[0, 0] + EPS
    q = q_ref[...]
    q2 = q + q
    qn = jnp.sum(q * q, axis=1, keepdims=True)

    def pass_1(c, z):
        xc = x_ref[:, pl.ds(c * CHUNK, CHUNK)]
        dot2 = jax.lax.dot_general(
            q2, xc, (((1,), (0,)), ((), ())),
            preferred_element_type=jnp.float32)
        neg_d = dot2 - (qn + xn_ref[:, pl.ds(c * CHUNK, CHUNK)])
        e = jnp.exp(neg_d / temp)
        e_ref[:, pl.ds(c * CHUNK, CHUNK)] = e
        return z + jnp.sum(e, axis=1, keepdims=True)

    z = jax.lax.fori_loop(0, NCHUNK, pass_1, jnp.zeros((QB, 1), jnp.float32))
    th = 0.001 * z

    def pass_2(c, acc):
        e = e_ref[:, pl.ds(c * CHUNK, CHUNK)]
        we = jnp.where(e >= th, e, 0.0)
        mc = mc_ref[:, pl.ds(c * CHUNK, CHUNK)]
        return acc + jnp.sum(we * mc, axis=1, keepdims=True)

    acc = jax.lax.fori_loop(0, NCHUNK, pass_2, jnp.zeros((QB, 1), jnp.float32))
    o_ref[...] = acc / z


@jax.jit
def kernel(query_vectors, temperature, indices, s1_metric):
    orig_dtype = query_vectors.dtype
    batch = query_vectors.shape[0]
    q = query_vectors.astype(jnp.float32)
    x_t = indices.astype(jnp.float32).T
    s = s1_metric.astype(jnp.float32).reshape(1, DB)
    t = temperature.astype(jnp.float32).reshape(1, 1)

    grid = (batch // QB,)
    out = pl.pallas_call(
        _lookup_kernel,
        grid=grid,
        in_specs=[
            pl.BlockSpec((QB, 3), lambda i: (i, 0)),
            pl.BlockSpec((1, 1), lambda i: (0, 0)),
            pl.BlockSpec((3, DB), lambda i: (0, 0)),
            pl.BlockSpec((1, DB), lambda i: (0, 0)),
        ],
        out_specs=pl.BlockSpec((QB, 1), lambda i: (i, 0)),
        out_shape=jax.ShapeDtypeStruct((batch, 1), jnp.float32),
        scratch_shapes=[
            pltpu.VMEM((QB, DB), jnp.float32),
            pltpu.VMEM((1, DB), jnp.float32),
            pltpu.VMEM((1, DB), jnp.float32),
        ],
    )(q, t, x_t, s)
    return out.reshape(batch).astype(orig_dtype)


# speedup vs baseline: 1.2071x; 1.0382x over previous
import jax
import jax.numpy as jnp
from jax.experimental import pallas as pl
from jax.experimental.pallas import tpu as pltpu

EPS = 1e-8

DB = 65536
QB = 128
CHUNK = 32768
NCHUNK = DB // CHUNK


def _lookup_kernel(q_ref, t_ref, x_ref, s_ref, o_ref, e_ref, xn_ref, mc_ref):
    @pl.when(pl.program_id(0) == 0)
    def _prologue():
        def norms_minmax(c, carry):
            mn, mx = carry
            xc = x_ref[:, pl.ds(c * CHUNK, CHUNK)]
            xn_ref[:, pl.ds(c * CHUNK, CHUNK)] = jnp.sum(
                xc * xc, axis=0, keepdims=True)
            sc = s_ref[:, pl.ds(c * CHUNK, CHUNK)]
            return jnp.minimum(mn, jnp.min(sc)), jnp.maximum(mx, jnp.max(sc))

        mn, mx = jax.lax.fori_loop(
            0, NCHUNK, norms_minmax, (jnp.inf, -jnp.inf))
        rng = mx - mn

        def norm_metric(c, _):
            sc = s_ref[:, pl.ds(c * CHUNK, CHUNK)]
            mc_ref[:, pl.ds(c * CHUNK, CHUNK)] = (sc - mn) / rng
            return 0

        jax.lax.fori_loop(0, NCHUNK, norm_metric, 0)

    temp = t_ref[0, 0] + EPS
    q = q_ref[...]
    q2 = q + q
    qn = jnp.sum(q * q, axis=1, keepdims=True)

    def pass_1(c, z):
        xc = x_ref[:, pl.ds(c * CHUNK, CHUNK)]
        dot2 = jax.lax.dot_general(
            q2, xc, (((1,), (0,)), ((), ())),
            preferred_element_type=jnp.float32)
        neg_d = dot2 - (qn + xn_ref[:, pl.ds(c * CHUNK, CHUNK)])
        e = jnp.exp(neg_d / temp)
        e_ref[:, pl.ds(c * CHUNK, CHUNK)] = e
        return z + jnp.sum(e, axis=1, keepdims=True)

    z = jax.lax.fori_loop(0, NCHUNK, pass_1, jnp.zeros((QB, 1), jnp.float32))
    th = 0.001 * z

    def pass_2(c, acc):
        e = e_ref[:, pl.ds(c * CHUNK, CHUNK)]
        we = jnp.where(e >= th, e, 0.0)
        mc = mc_ref[:, pl.ds(c * CHUNK, CHUNK)]
        return acc + jnp.sum(we * mc, axis=1, keepdims=True)

    acc = jax.lax.fori_loop(0, NCHUNK, pass_2, jnp.zeros((QB, 1), jnp.float32))
    o_ref[...] = acc / z


@jax.jit
def kernel(query_vectors, temperature, indices, s1_metric):
    orig_dtype = query_vectors.dtype
    batch = query_vectors.shape[0]
    q = query_vectors.astype(jnp.float32)
    x_t = indices.astype(jnp.float32).T
    s = s1_metric.astype(jnp.float32).reshape(1, DB)
    t = temperature.astype(jnp.float32).reshape(1, 1)

    grid = (batch // QB,)
    out = pl.pallas_call(
        _lookup_kernel,
        grid=grid,
        in_specs=[
            pl.BlockSpec((QB, 3), lambda i: (i, 0)),
            pl.BlockSpec((1, 1), lambda i: (0, 0)),
            pl.BlockSpec((3, DB), lambda i: (0, 0)),
            pl.BlockSpec((1, DB), lambda i: (0, 0)),
        ],
        out_specs=pl.BlockSpec((QB, 1), lambda i: (i, 0)),
        out_shape=jax.ShapeDtypeStruct((batch, 1), jnp.float32),
        scratch_shapes=[
            pltpu.VMEM((QB, DB), jnp.float32),
            pltpu.VMEM((1, DB), jnp.float32),
            pltpu.VMEM((1, DB), jnp.float32),
        ],
    )(q, t, x_t, s)
    return out.reshape(batch).astype(orig_dtype)
